# baseline (device time: 169522 ns/iter reference)
import jax
import jax.numpy as jnp
from jax import lax
from jax.experimental import pallas as pl
from jax.experimental.pallas import tpu as pltpu

N_DEV = 4


def kernel(x, w_mat, scale_x, scale_w):
    if x.dtype != jnp.float8_e5m2:
        x = x.astype(jnp.float8_e5m2)
    if w_mat.dtype != jnp.float8_e5m2:
        w_mat = w_mat.astype(jnp.float8_e5m2)
    scale_x = scale_x.astype(jnp.float32)
    scale_w = scale_w.astype(jnp.float32)

    m_per, k = x.shape
    _, n_per = w_mat.shape

    def body(x_ref, w_ref, sx_ref, sw_ref, out_ref, comm_ref, send_sems, recv_sems):
        my = lax.axis_index("i")
        left = lax.rem(my + (N_DEV - 1), N_DEV)
        right = lax.rem(my + 1, N_DEV)

        barrier_sem = pltpu.get_barrier_semaphore()
        for nbr in (left, right):
            pl.semaphore_signal(
                barrier_sem, inc=1,
                device_id=(nbr,), device_id_type=pl.DeviceIdType.MESH,
            )
        pl.semaphore_wait(barrier_sem, 2)

        scale = sx_ref[0] * sw_ref[0]

        def gemm_store(chunk, origin):
            acc = jnp.dot(chunk, w_ref[...], preferred_element_type=jnp.float32)
            y = acc * scale
            z = jnp.clip(y, -60.0, 60.0)
            out_ref[pl.ds(origin * m_per, m_per), :] = y * (
                1.0 / (1.0 + jnp.exp(-z))
            )

        comm_ref[0] = x_ref[...]
        gemm_store(x_ref[...], my)

        for h in range(N_DEV - 1):
            send_slot = h % 2
            recv_slot = (h + 1) % 2
            rdma = pltpu.make_async_remote_copy(
                src_ref=comm_ref.at[send_slot],
                dst_ref=comm_ref.at[recv_slot],
                send_sem=send_sems.at[send_slot],
                recv_sem=recv_sems.at[recv_slot],
                device_id=(right,),
                device_id_type=pl.DeviceIdType.MESH,
            )
            rdma.start()
            rdma.wait()
            origin = lax.rem(my + (N_DEV - 1 - h), N_DEV)
            gemm_store(comm_ref[recv_slot], origin)

    return pl.pallas_call(
        body,
        out_shape=jax.ShapeDtypeStruct((N_DEV * m_per, n_per), jnp.float32),
        in_specs=[
            pl.BlockSpec(memory_space=pltpu.VMEM),
            pl.BlockSpec(memory_space=pltpu.VMEM),
            pl.BlockSpec(memory_space=pltpu.SMEM),
            pl.BlockSpec(memory_space=pltpu.SMEM),
        ],
        out_specs=pl.BlockSpec(memory_space=pltpu.VMEM),
        scratch_shapes=[
            pltpu.VMEM((2, m_per, k), jnp.float8_e5m2),
            pltpu.SemaphoreType.DMA((2,)),
            pltpu.SemaphoreType.DMA((2,)),
        ],
        compiler_params=pltpu.CompilerParams(collective_id=0),
    )(x, w_mat, scale_x, scale_w)


# device time: 92615 ns/iter; 1.8304x vs baseline; 1.8304x over previous
import jax
import jax.numpy as jnp
from jax import lax
from jax.experimental import pallas as pl
from jax.experimental.pallas import tpu as pltpu

N_DEV = 4


def kernel(x, w_mat, scale_x, scale_w):
    if x.dtype != jnp.float8_e5m2:
        x = x.astype(jnp.float8_e5m2)
    if w_mat.dtype != jnp.float8_e5m2:
        w_mat = w_mat.astype(jnp.float8_e5m2)
    scale_x = scale_x.astype(jnp.float32)
    scale_w = scale_w.astype(jnp.float32)

    m_per, k = x.shape
    _, n_per = w_mat.shape
    half = m_per // 2

    def body(x_ref, w_ref, sx_ref, sw_ref, out_ref,
             rl_buf, rr_buf, hl_buf, hr_buf, send_sems, recv_sems):
        my = lax.axis_index("i")
        left = lax.rem(my + (N_DEV - 1), N_DEV)
        right = lax.rem(my + 1, N_DEV)

        barrier_sem = pltpu.get_barrier_semaphore()
        for nbr in (left, right):
            pl.semaphore_signal(
                barrier_sem, inc=1,
                device_id=(nbr,), device_id_type=pl.DeviceIdType.MESH,
            )
        pl.semaphore_wait(barrier_sem, 2)

        s1r = pltpu.make_async_remote_copy(
            src_ref=x_ref, dst_ref=rl_buf,
            send_sem=send_sems.at[0], recv_sem=recv_sems.at[0],
            device_id=(right,), device_id_type=pl.DeviceIdType.MESH,
        )
        s1l = pltpu.make_async_remote_copy(
            src_ref=x_ref, dst_ref=rr_buf,
            send_sem=send_sems.at[1], recv_sem=recv_sems.at[1],
            device_id=(left,), device_id_type=pl.DeviceIdType.MESH,
        )
        s1r.start()
        s1l.start()

        scale = sx_ref[0] * sw_ref[0]

        def gemm_store(chunk, row_start):
            acc = jnp.dot(chunk, w_ref[...], preferred_element_type=jnp.float32)
            y = acc * scale
            z = jnp.clip(y, -60.0, 60.0)
            out_ref[pl.ds(row_start, chunk.shape[0]), :] = y * (
                1.0 / (1.0 + jnp.exp(-z))
            )

        gemm_store(x_ref[...], my * m_per)

        s1r.wait_recv()
        s2r = pltpu.make_async_remote_copy(
            src_ref=rl_buf.at[pl.ds(0, half)], dst_ref=hl_buf,
            send_sem=send_sems.at[2], recv_sem=recv_sems.at[2],
            device_id=(right,), device_id_type=pl.DeviceIdType.MESH,
        )
        s2r.start()
        gemm_store(rl_buf[...], left * m_per)

        s1l.wait_recv()
        s2l = pltpu.make_async_remote_copy(
            src_ref=rr_buf.at[pl.ds(half, half)], dst_ref=hr_buf,
            send_sem=send_sems.at[3], recv_sem=recv_sems.at[3],
            device_id=(left,), device_id_type=pl.DeviceIdType.MESH,
        )
        s2l.start()
        gemm_store(rr_buf[...], right * m_per)

        opp_row = lax.rem(my + 2, N_DEV) * m_per
        s2r.wait_recv()
        gemm_store(hl_buf[...], opp_row)
        s2l.wait_recv()
        gemm_store(hr_buf[...], opp_row + half)

        s1r.wait_send()
        s1l.wait_send()
        s2r.wait_send()
        s2l.wait_send()

    return pl.pallas_call(
        body,
        out_shape=jax.ShapeDtypeStruct((N_DEV * m_per, n_per), jnp.float32),
        in_specs=[
            pl.BlockSpec(memory_space=pltpu.VMEM),
            pl.BlockSpec(memory_space=pltpu.VMEM),
            pl.BlockSpec(memory_space=pltpu.SMEM),
            pl.BlockSpec(memory_space=pltpu.SMEM),
        ],
        out_specs=pl.BlockSpec(memory_space=pltpu.VMEM),
        scratch_shapes=[
            pltpu.VMEM((m_per, k), jnp.float8_e5m2),
            pltpu.VMEM((m_per, k), jnp.float8_e5m2),
            pltpu.VMEM((half, k), jnp.float8_e5m2),
            pltpu.VMEM((half, k), jnp.float8_e5m2),
            pltpu.SemaphoreType.DMA((4,)),
            pltpu.SemaphoreType.DMA((4,)),
        ],
        compiler_params=pltpu.CompilerParams(collective_id=0),
    )(x, w_mat, scale_x, scale_w)
